# baseline (speedup 1.0000x reference)
import functools

import jax
import jax.numpy as jnp
from jax.experimental import pallas as pl
from jax.experimental.pallas import tpu as pltpu

N = 16384
BLK = 256
NBLK = N // BLK


def _cox_kernel(x_row, t_row, e_row, t_col, e_col, out_ref):
    x = x_row[...]
    t = t_row[...]
    e = e_row[...]
    m = jnp.max(x)
    w = jnp.exp(x - m)
    term1 = jnp.sum(x * e)

    def body(b, acc):
        thr = t_col[pl.ds(b * BLK, BLK), :]
        ev = e_col[pl.ds(b * BLK, BLK), :]
        mask = (t >= thr).astype(jnp.float32)
        s = jax.lax.dot_general(
            mask, w, (((1,), (1,)), ((), ())),
            preferred_element_type=jnp.float32,
            precision=jax.lax.Precision.HIGHEST)
        return acc + jnp.sum(ev * (m + jnp.log(s)))

    acc = jax.lax.fori_loop(0, NBLK, body, jnp.float32(0.0))
    loss = (acc - term1) / N
    out_ref[...] = jnp.sqrt(loss)[None, None]


@jax.jit
def kernel(input, target):
    x = input.reshape(1, N)
    t = target[:, 0]
    e = target[:, 1]
    out = pl.pallas_call(
        _cox_kernel,
        out_shape=jax.ShapeDtypeStruct((1, 1), jnp.float32),
        in_specs=[
            pl.BlockSpec((1, N), lambda: (0, 0)),
            pl.BlockSpec((1, N), lambda: (0, 0)),
            pl.BlockSpec((1, N), lambda: (0, 0)),
            pl.BlockSpec((N, 1), lambda: (0, 0)),
            pl.BlockSpec((N, 1), lambda: (0, 0)),
        ],
        out_specs=pl.BlockSpec((1, 1), lambda: (0, 0)),
    )(x, t.reshape(1, N), e.reshape(1, N), t.reshape(N, 1), e.reshape(N, 1))
    return out[0, 0]


# speedup vs baseline: 2.2207x; 2.2207x over previous
import functools

import jax
import jax.numpy as jnp
from jax import lax
from jax.experimental import pallas as pl
from jax.experimental.pallas import tpu as pltpu
from jax.experimental.pallas import tpu_sc as plsc

N = 16384
NT = 16
CH = N // NT
LPL = CH // 16
RADIX = 32
KMAX = 0x3F7FFFFF

@functools.cache
def _get_sc_sort():
    mesh = plsc.VectorSubcoreMesh(core_axis_name="c", subcore_axis_name="s")
    return pl.kernel(
        _sc_sort_body,
        out_type=[jax.ShapeDtypeStruct((N,), jnp.float32)] * 3,
        mesh=mesh,
        compiler_params=pltpu.CompilerParams(needs_layout_passes=False),
        scratch_types=[
            pltpu.VMEM((CH,), jnp.int32),
            pltpu.VMEM((CH,), jnp.int32),
            pltpu.VMEM((8, 128), jnp.int32),
            pltpu.VMEM((512,), jnp.int32),
            pltpu.VMEM((512,), jnp.int32),
            pltpu.VMEM((32,), jnp.int32),
            pltpu.VMEM((512,), jnp.int32),
            pltpu.VMEM((8, 128), jnp.int32),
            pltpu.VMEM((8, 128), jnp.float32),
            pltpu.VMEM_SHARED((N,), jnp.int32),
            pltpu.VMEM_SHARED((N,), jnp.int32),
            pltpu.VMEM_SHARED((N,), jnp.int32),
            pltpu.VMEM_SHARED((N,), jnp.int32),
            pltpu.VMEM_SHARED((512,), jnp.int32),
        ],
    )


def _sc_sort_body(k_hbm, t_hbm, x_hbm, e_hbm, ts_out, xs_out, es_out,
             kv, iv, pv, bv2, bzv, h32v, hall, iv2, gv,
             KA, KB, IA, IB, HALL):
    cid = lax.axis_index("c")
    tid = lax.axis_index("s")

    @pl.when(cid == 0)
    def _body():
        base = tid * CH
        lane = lax.iota(jnp.int32, 16)
        ones = jnp.ones((16,), jnp.int32)
        zeros16 = jnp.zeros((16,), jnp.int32)

        pltpu.sync_copy(k_hbm.at[pl.ds(base, CH)], kv)
        def stage_body(s, carry):
            iv[pl.ds(s * 16, 16)] = base + s * 16 + lane
            return carry
        lax.fori_loop(0, CH // 16, stage_body, 0)
        pltpu.sync_copy(kv, KA.at[pl.ds(base, CH)])
        pltpu.sync_copy(iv, IA.at[pl.ds(base, CH)])
        plsc.subcore_barrier()

        def one_pass(shift, src_k, src_i, dst_k, dst_i):
            pltpu.sync_copy(src_k.at[pl.ds(base, CH)], kv)
            pltpu.sync_copy(src_i.at[pl.ds(base, CH)], iv)
            for s in range(32):
                bv2[pl.ds(s * 16, 16)] = zeros16
            def hist_body(v, carry):
                bidx = lane * LPL + v
                k = plsc.load_gather(kv, [bidx])
                d = lax.shift_right_logical(k, shift) & (RADIX - 1)
                plsc.addupdate_scatter(bv2, [lane * RADIX + d], ones)
                return carry
            lax.fori_loop(0, LPL, hist_body, 0)
            lo = zeros16
            hi = zeros16
            for l in range(16):
                lo = lo + bv2[pl.ds(l * 32, 16)]
                hi = hi + bv2[pl.ds(l * 32 + 16, 16)]
            h32v[pl.ds(0, 16)] = lo
            h32v[pl.ds(16, 16)] = hi
            pltpu.sync_copy(h32v, HALL.at[pl.ds(tid * 32, 32)])
            plsc.subcore_barrier()
            pltpu.sync_copy(HALL, hall)
            tot_lo = zeros16
            tot_hi = zeros16
            te_lo = zeros16
            te_hi = zeros16
            for tp in range(16):
                row_lo = hall[pl.ds(tp * 32, 16)]
                row_hi = hall[pl.ds(tp * 32 + 16, 16)]
                tot_lo = tot_lo + row_lo
                tot_hi = tot_hi + row_hi
                sel = jnp.where(tp < tid, 1, 0).astype(jnp.int32)
                te_lo = te_lo + row_lo * sel
                te_hi = te_hi + row_hi * sel
            c_lo = plsc.cumsum(tot_lo) - tot_lo
            s_lo = jnp.sum(tot_lo)
            c_hi = plsc.cumsum(tot_hi) - tot_hi + s_lo
            off_lo = c_lo + te_lo
            off_hi = c_hi + te_hi
            run_lo = off_lo
            run_hi = off_hi
            for l in range(16):
                bzv[pl.ds(l * 32, 16)] = run_lo
                bzv[pl.ds(l * 32 + 16, 16)] = run_hi
                run_lo = run_lo + bv2[pl.ds(l * 32, 16)]
                run_hi = run_hi + bv2[pl.ds(l * 32 + 16, 16)]
            def rank_body(v, carry):
                bidx = lane * LPL + v
                k = plsc.load_gather(kv, [bidx])
                d = lax.shift_right_logical(k, shift) & (RADIX - 1)
                bin_ = lane * RADIX + d
                pos = plsc.load_gather(bzv, [bin_])
                plsc.addupdate_scatter(bzv, [bin_], ones)
                plsc.store_scatter(
                    pv, [lax.shift_right_logical(bidx, 7), bidx & 127], pos)
                return carry
            lax.fori_loop(0, LPL, rank_body, 0)
            for j in range(8):
                pltpu.sync_copy(kv.at[pl.ds(j * 128, 128)], dst_k.at[pv.at[j]])
                pltpu.sync_copy(iv.at[pl.ds(j * 128, 128)], dst_i.at[pv.at[j]])
            plsc.subcore_barrier()

        one_pass(0, KA, IA, KB, IB)
        one_pass(5, KB, IB, KA, IA)
        one_pass(10, KA, IA, KB, IB)
        one_pass(15, KB, IB, KA, IA)
        one_pass(20, KA, IA, KB, IB)
        one_pass(25, KB, IB, KA, IA)

        for j in range(8):
            pltpu.sync_copy(IA.at[pl.ds(base + j * 128, 128)], iv2.at[j])
        for j in range(8):
            pltpu.sync_copy(t_hbm.at[iv2.at[j]], gv.at[j])
        for j in range(8):
            pltpu.sync_copy(gv.at[j], ts_out.at[pl.ds(base + j * 128, 128)])
        for j in range(8):
            pltpu.sync_copy(x_hbm.at[iv2.at[j]], gv.at[j])
        for j in range(8):
            pltpu.sync_copy(gv.at[j], xs_out.at[pl.ds(base + j * 128, 128)])
        for j in range(8):
            pltpu.sync_copy(e_hbm.at[iv2.at[j]], gv.at[j])
        for j in range(8):
            pltpu.sync_copy(gv.at[j], es_out.at[pl.ds(base + j * 128, 128)])


def _tc_finish(ts_ref, xs_ref, es_ref, out_ref):
    ts = ts_ref[...]
    xs = xs_ref[...]
    es = es_ref[...]
    m = jnp.max(xs)
    ws = jnp.exp(xs - m)
    r = lax.broadcasted_iota(jnp.int32, (128, 128), 0)
    c = lax.broadcasted_iota(jnp.int32, (128, 128), 1)
    up = (r <= c).astype(jnp.float32)
    lo = (r > c).astype(jnp.float32)
    rowcs = lax.dot_general(ws, up, (((1,), (0,)), ((), ())),
                            preferred_element_type=jnp.float32,
                            precision=lax.Precision.HIGHEST)
    rowtot = rowcs[:, 127:128]
    offs = lax.dot_general(lo, rowtot, (((1,), (0,)), ((), ())),
                           preferred_element_type=jnp.float32,
                           precision=lax.Precision.HIGHEST)
    p = rowcs + offs
    q = p
    for d in (1, 2, 4, 8, 16, 32, 64):
        ts_d = jnp.concatenate(
            [ts[:, d:],
             jnp.concatenate([ts[1:, :d], jnp.full((1, d), -1.0, jnp.float32)],
                             axis=0)], axis=1)
        q_d = jnp.concatenate(
            [q[:, d:],
             jnp.concatenate([q[1:, :d], jnp.ones((1, d), jnp.float32)],
                             axis=0)], axis=1)
        q = jnp.where(ts_d == ts, q_d, q)
    for dm in (1, 2, 4, 8, 16, 32, 64):
        ts_d = jnp.concatenate(
            [ts[dm:, :], jnp.full((dm, 128), -1.0, jnp.float32)], axis=0)
        q_d = jnp.concatenate(
            [q[dm:, :], jnp.ones((dm, 128), jnp.float32)], axis=0)
        q = jnp.where(ts_d == ts, q_d, q)
    loss2 = jnp.sum(es * (m + jnp.log(q)))
    term1 = jnp.sum(xs * es)
    out_ref[...] = jnp.sqrt((loss2 - term1) / N)[None, None]


@jax.jit
def kernel(input, target):
    t = target[:, 0]
    e = target[:, 1]
    k = KMAX - lax.bitcast_convert_type(t, jnp.int32)
    ts, xs, es = _get_sc_sort()(k, t, input, e)
    out = pl.pallas_call(
        _tc_finish,
        out_shape=jax.ShapeDtypeStruct((1, 1), jnp.float32),
        in_specs=[
            pl.BlockSpec((128, 128), lambda: (0, 0)),
            pl.BlockSpec((128, 128), lambda: (0, 0)),
            pl.BlockSpec((128, 128), lambda: (0, 0)),
        ],
        out_specs=pl.BlockSpec((1, 1), lambda: (0, 0)),
    )(ts.reshape(128, 128), xs.reshape(128, 128), es.reshape(128, 128))
    return out[0, 0]
